# baseline (speedup 1.0000x reference)
import functools

import jax
import jax.numpy as jnp
from jax import lax
from jax.experimental import pallas as pl
from jax.experimental.pallas import tpu as pltpu
from jax.experimental.pallas import tpu_sc as plsc

N1 = 10000
N2 = 2000
T = 12
F0 = 128
C = 128
SECOND = 16

NTILES = 32
_K = 128


def _sc_mesh():
    return plsc.VectorSubcoreMesh(
        core_axis_name="c", subcore_axis_name="s", num_cores=2, num_subcores=16
    )


@functools.lru_cache(maxsize=None)
def _make_hist(e_pad, nh):
    per_tile = e_pad // NTILES
    assert per_tile % 16 == 0 and nh % 16 == 0

    @functools.partial(
        pl.kernel,
        out_type=jax.ShapeDtypeStruct((NTILES, nh), jnp.float32),
        mesh=_sc_mesh(),
        scratch_types=[
            pltpu.VMEM((per_tile,), jnp.int32),
            pltpu.VMEM((nh,), jnp.float32),
        ],
        compiler_params=pltpu.CompilerParams(needs_layout_passes=False),
    )
    def hist(lab_hbm, out_hbm, lab_v, hist_v):
        c = lax.axis_index("c")
        s = lax.axis_index("s")
        wid = c * 16 + s
        zvec = jnp.zeros((16,), jnp.float32)

        def zbody(i, carry):
            hist_v[pl.ds(i * 16, 16)] = zvec
            return carry

        lax.fori_loop(0, nh // 16, zbody, 0)
        pltpu.sync_copy(lab_hbm.at[pl.ds(wid * per_tile, per_tile)], lab_v)
        ones = jnp.full((16,), 1.0, jnp.float32)

        def body(i, carry):
            idx = lab_v[pl.ds(i * 16, 16)]
            plsc.addupdate_scatter(hist_v, [idx], ones)
            return carry

        lax.fori_loop(0, per_tile // 16, body, 0)
        pltpu.sync_copy(hist_v, out_hbm.at[wid])

    return hist


@functools.lru_cache(maxsize=None)
def _make_segsum(e_pad, n_acc, f):
    per_tile = e_pad // NTILES
    nchunks = per_tile // _K
    npairs = nchunks // 2
    rpt = n_acc // 16
    assert per_tile % (2 * _K) == 0 and rpt % 16 == 0 and f % 16 == 0

    @functools.partial(
        pl.kernel,
        out_type=jax.ShapeDtypeStruct((2, n_acc, f), jnp.float32),
        mesh=_sc_mesh(),
        scratch_types=[
            pltpu.VMEM((_K,), jnp.int32),
            pltpu.VMEM((_K,), jnp.int32),
            pltpu.VMEM((_K,), jnp.int32),
            pltpu.VMEM((_K,), jnp.int32),
            pltpu.VMEM((_K, f), jnp.float32),
            pltpu.VMEM((_K, f), jnp.float32),
            pltpu.VMEM((16, f), jnp.float32),
            pltpu.VMEM_SHARED((n_acc, f), jnp.float32),
            pltpu.SemaphoreType.DMA,
            pltpu.SemaphoreType.DMA,
            pltpu.SemaphoreType.DMA,
            pltpu.SemaphoreType.DMA,
        ],
    )
    def seg(src_hbm, dst_hbm, table_hbm, out_hbm, sa_i, da_i, sb_i, db_i,
            rows_a, rows_b, zb, acc, sem_a, sem_b, sem_sa, sem_sb):
        c = lax.axis_index("c")
        s = lax.axis_index("s")
        wid = c * 16 + s
        zvec = jnp.zeros((16,), jnp.float32)
        for r in range(16):
            for q in range(f // 16):
                zb[r, pl.ds(q * 16, 16)] = zvec

        base = wid * per_tile
        pltpu.sync_copy(src_hbm.at[pl.ds(base, _K)], sa_i)
        pltpu.sync_copy(dst_hbm.at[pl.ds(base, _K)], da_i)
        pltpu.async_copy(table_hbm.at[sa_i], rows_a, sem_a)

        def zbody(i, carry):
            pltpu.sync_copy(zb, acc.at[pl.ds(s * rpt + i * 16, 16)])
            return carry

        lax.fori_loop(0, rpt // 16, zbody, 0)
        plsc.subcore_barrier()

        def pair(j, carry):
            i0 = 2 * j
            pltpu.make_async_copy(table_hbm.at[sa_i], rows_a, sem_a).wait()
            pltpu.async_copy(rows_a, acc.at[da_i], sem_sa, add=True)

            @pl.when(j > 0)
            def _():
                pltpu.make_async_copy(rows_b, acc.at[db_i], sem_sb).wait()

            pltpu.sync_copy(src_hbm.at[pl.ds(base + (i0 + 1) * _K, _K)], sb_i)
            pltpu.sync_copy(dst_hbm.at[pl.ds(base + (i0 + 1) * _K, _K)], db_i)
            pltpu.async_copy(table_hbm.at[sb_i], rows_b, sem_b)
            pltpu.make_async_copy(table_hbm.at[sb_i], rows_b, sem_b).wait()
            pltpu.async_copy(rows_b, acc.at[db_i], sem_sb, add=True)

            pltpu.make_async_copy(rows_a, acc.at[da_i], sem_sa).wait()
            inext = lax.rem(i0 + 2, nchunks)
            pltpu.sync_copy(src_hbm.at[pl.ds(base + inext * _K, _K)], sa_i)
            pltpu.sync_copy(dst_hbm.at[pl.ds(base + inext * _K, _K)], da_i)
            pltpu.async_copy(table_hbm.at[sa_i], rows_a, sem_a)
            return carry

        lax.fori_loop(0, npairs, pair, 0)
        pltpu.make_async_copy(table_hbm.at[sa_i], rows_a, sem_a).wait()
        pltpu.make_async_copy(rows_b, acc.at[db_i], sem_sb).wait()
        plsc.subcore_barrier()
        pltpu.sync_copy(
            acc.at[pl.ds(s * rpt, rpt)], out_hbm.at[c].at[pl.ds(s * rpt, rpt)]
        )

    return seg


def _pad_idx(a, e_pad, fill):
    return jnp.concatenate(
        [a, jnp.full((e_pad - a.shape[0],), fill, jnp.int32)]
    )


def _epad(e):
    blk = NTILES * _K * 2
    return ((e + blk - 1) // blk) * blk


def _epad_hist(e):
    per = NTILES * 16
    return ((e + per - 1) // per) * per


def _segsum(src, dst, table, n_acc, dummy):
    e = src.shape[0]
    e_pad = _epad(e)
    srcp = _pad_idx(src, e_pad, 0)
    dstp = _pad_idx(dst, e_pad, dummy)
    return _make_segsum(e_pad, n_acc, table.shape[1])(srcp, dstp, table)


def _tc_colsum(hist, n, mode):
    def body(h_ref, o_ref):
        ssum = jnp.sum(h_ref[...], axis=0, keepdims=True)
        if mode == "dinv":
            o_ref[...] = lax.rsqrt(ssum[:, :n] + 1.0)
        else:
            o_ref[...] = 1.0 / jnp.maximum(ssum[:, :n], 1e-12)

    out = pl.pallas_call(
        body, out_shape=jax.ShapeDtypeStruct((1, n), jnp.float32)
    )(hist)
    return out.reshape(n, 1)


def _tc_scale3(xt, d, bn):
    n = xt.shape[1]

    def body(x_ref, d_ref, o_ref):
        o_ref[...] = x_ref[...] * d_ref[...][None]

    return pl.pallas_call(
        body,
        grid=(T, n // bn),
        in_specs=[
            pl.BlockSpec((1, bn, F0), lambda t, j: (t, j, 0)),
            pl.BlockSpec((bn, 1), lambda t, j: (j, 0)),
        ],
        out_specs=pl.BlockSpec((1, bn, F0), lambda t, j: (t, j, 0)),
        out_shape=jax.ShapeDtypeStruct(xt.shape, jnp.float32),
    )(xt, d)


def _tc_scale2(tbl, d, bn):
    n, f = tbl.shape

    def body(x_ref, d_ref, o_ref):
        o_ref[...] = x_ref[...] * d_ref[...]

    return pl.pallas_call(
        body,
        grid=(n // bn,),
        in_specs=[
            pl.BlockSpec((bn, f), lambda j: (j, 0)),
            pl.BlockSpec((bn, 1), lambda j: (j, 0)),
        ],
        out_specs=pl.BlockSpec((bn, f), lambda j: (j, 0)),
        out_shape=jax.ShapeDtypeStruct((n, f), jnp.float32),
    )(tbl, d)


def _tc_main1(xts, plist, d, wz, bz, wh, bh, probs, bn):
    n = xts.shape[1]

    def body(*refs):
        x_ref = refs[0]
        p_refs = refs[1 : 1 + T]
        d_ref, wz_ref, bz_ref, wh_ref, bh_ref, pr_ref, o_ref, st_ref = refs[1 + T :]
        dv = d_ref[...]
        acc = jnp.zeros((bn, C), jnp.float32)
        cols = []
        for t in range(T):
            S = (p_refs[t][0] + p_refs[t][1] + x_ref[t]) * dv
            z = jax.nn.sigmoid(
                jnp.dot(S, wz_ref[...], preferred_element_type=jnp.float32)
                + bz_ref[...]
            )
            ht = jnp.tanh(
                jnp.dot(S, wh_ref[...], preferred_element_type=jnp.float32)
                + bh_ref[...]
            )
            acc = acc + pr_ref[0, t] * (1.0 - z) * ht
            cols.append(S[:, C - 1 : C])
        o_ref[...] = jnp.maximum(acc, 0.0)
        cols.append(jnp.zeros((bn, 16 - T), jnp.float32))
        st_ref[...] = jnp.concatenate(cols, axis=1)

    in_specs = [pl.BlockSpec((T, bn, F0), lambda j: (0, j, 0))]
    in_specs += [pl.BlockSpec((2, bn, C), lambda j: (0, j, 0)) for _ in range(T)]
    in_specs += [
        pl.BlockSpec((bn, 1), lambda j: (j, 0)),
        pl.BlockSpec((F0, C), lambda j: (0, 0)),
        pl.BlockSpec((1, C), lambda j: (0, 0)),
        pl.BlockSpec((F0, C), lambda j: (0, 0)),
        pl.BlockSpec((1, C), lambda j: (0, 0)),
        pl.BlockSpec((1, 16), lambda j: (0, 0)),
    ]
    return pl.pallas_call(
        body,
        grid=(n // bn,),
        in_specs=in_specs,
        out_specs=[
            pl.BlockSpec((bn, C), lambda j: (j, 0)),
            pl.BlockSpec((bn, 16), lambda j: (j, 0)),
        ],
        out_shape=[
            jax.ShapeDtypeStruct((n, C), jnp.float32),
            jax.ShapeDtypeStruct((n, 16), jnp.float32),
        ],
    )(xts, *plist, d, wz, bz, wh, bh, probs)


def _tc_main2(sp, tbl, st, d, wzh, bz, whh, bh, wzl, whl, probs, bn):
    n, f = tbl.shape

    def body(sp_ref, t_ref, st_ref, d_ref, wzh_r, bz_r, whh_r, bh_r,
             wzl_r, whl_r, pr_r, o_ref):
        Sh = (sp_ref[0] + sp_ref[1] + t_ref[...]) * d_ref[...]
        St = st_ref[...]
        Pz = jnp.dot(Sh, wzh_r[...], preferred_element_type=jnp.float32) + bz_r[...]
        Ph = jnp.dot(Sh, whh_r[...], preferred_element_type=jnp.float32) + bh_r[...]
        acc = jnp.zeros((bn, C), jnp.float32)
        for t in range(T):
            z = jax.nn.sigmoid(Pz + St[:, t : t + 1] * wzl_r[...])
            ht = jnp.tanh(Ph + St[:, t : t + 1] * whl_r[...])
            acc = acc + pr_r[0, t] * (1.0 - z) * ht
        o_ref[...] = jnp.maximum(acc, 0.0)

    in_specs = [
        pl.BlockSpec((2, bn, f), lambda j: (0, j, 0)),
        pl.BlockSpec((bn, f), lambda j: (j, 0)),
        pl.BlockSpec((bn, 16), lambda j: (j, 0)),
        pl.BlockSpec((bn, 1), lambda j: (j, 0)),
        pl.BlockSpec((C, C), lambda j: (0, 0)),
        pl.BlockSpec((1, C), lambda j: (0, 0)),
        pl.BlockSpec((C, C), lambda j: (0, 0)),
        pl.BlockSpec((1, C), lambda j: (0, 0)),
        pl.BlockSpec((1, C), lambda j: (0, 0)),
        pl.BlockSpec((1, C), lambda j: (0, 0)),
        pl.BlockSpec((1, 16), lambda j: (0, 0)),
    ]
    return pl.pallas_call(
        body,
        grid=(n // bn,),
        in_specs=in_specs,
        out_specs=pl.BlockSpec((bn, C), lambda j: (j, 0)),
        out_shape=jax.ShapeDtypeStruct((n, C), jnp.float32),
    )(sp, tbl, st, d, wzh, bz, whh, bh, wzl, whl, probs)


def _tc_main3(spA, spB, tblA, tblB, d, wzx, bz, whx, bh, wzl, whl, probs, bn):
    n = tblA.shape[0]

    def body(spA_r, spB_r, tA_r, tB_r, d_ref, wzx_r, bz_r, whx_r, bh_r,
             wzl_r, whl_r, pr_r, o_ref):
        dv = d_ref[...]
        SA = (spA_r[0] + spA_r[1] + tA_r[...]) * dv
        SB = (spB_r[0] + spB_r[1] + tB_r[...]) * dv
        Sh = jnp.concatenate([SA[:, :SECOND], SB], axis=1)
        St = SA[:, SECOND : SECOND + T]
        Pz = jnp.dot(Sh, wzx_r[...], preferred_element_type=jnp.float32) + bz_r[...]
        Ph = jnp.dot(Sh, whx_r[...], preferred_element_type=jnp.float32) + bh_r[...]
        acc = jnp.zeros((bn, C), jnp.float32)
        for t in range(T):
            z = jax.nn.sigmoid(Pz + St[:, t : t + 1] * wzl_r[...])
            ht = jnp.tanh(Ph + St[:, t : t + 1] * whl_r[...])
            acc = acc + pr_r[0, t] * (1.0 - z) * ht
        o_ref[...] = jnp.maximum(acc, 0.0)

    fm = SECOND + C
    in_specs = [
        pl.BlockSpec((2, bn, C), lambda j: (0, j, 0)),
        pl.BlockSpec((2, bn, C), lambda j: (0, j, 0)),
        pl.BlockSpec((bn, C), lambda j: (j, 0)),
        pl.BlockSpec((bn, C), lambda j: (j, 0)),
        pl.BlockSpec((bn, 1), lambda j: (j, 0)),
        pl.BlockSpec((fm, C), lambda j: (0, 0)),
        pl.BlockSpec((1, C), lambda j: (0, 0)),
        pl.BlockSpec((fm, C), lambda j: (0, 0)),
        pl.BlockSpec((1, C), lambda j: (0, 0)),
        pl.BlockSpec((1, C), lambda j: (0, 0)),
        pl.BlockSpec((1, C), lambda j: (0, 0)),
        pl.BlockSpec((1, 16), lambda j: (0, 0)),
    ]
    return pl.pallas_call(
        body,
        grid=(n // bn,),
        in_specs=in_specs,
        out_specs=pl.BlockSpec((bn, C), lambda j: (j, 0)),
        out_shape=jax.ShapeDtypeStruct((n, C), jnp.float32),
    )(spA, spB, tblA, tblB, d, wzx, bz, whx, bh, wzl, whl, probs)


def _tc_groupcombine(sa_h, sa_t, se_h, se_t, ra, re, x2, d2):
    def body(sah_r, sat_r, seh_r, set_r, ra_r, re_r, x2_r, d2_r, oa_ref, ob_ref):
        rav = ra_r[...]
        rev = re_r[...]
        gh = ((sah_r[0, :N2] + sah_r[1, :N2]) * rav
              + (seh_r[0, :N2] + seh_r[1, :N2]) * rev) * 0.5
        gt = ((sat_r[0, :N2] + sat_r[1, :N2]) * rav
              + (set_r[0, :N2] + set_r[1, :N2]) * rev) * 0.5
        d2v = d2_r[...]
        oa_ref[...] = (
            jnp.concatenate(
                [x2_r[...], gt[:, :T], jnp.zeros((N2, C - SECOND - T), jnp.float32)],
                axis=1,
            )
            * d2v
        )
        ob_ref[...] = gh * d2v

    return pl.pallas_call(
        body,
        out_shape=[
            jax.ShapeDtypeStruct((N2, C), jnp.float32),
            jax.ShapeDtypeStruct((N2, C), jnp.float32),
        ],
    )(sa_h, sa_t, se_h, se_t, ra, re, x2, d2)


def _tc_final(h1, g, w1a, w1b, b1, w2, b2, bn):
    n = h1.shape[0]

    def body(h1_r, g_r, w1a_r, w1b_r, b1_r, w2_r, b2_r, o_ref):
        h2g = (g_r[0] + g_r[1]) * 0.5
        hh = jnp.maximum(
            jnp.dot(h1_r[...], w1a_r[...], preferred_element_type=jnp.float32)
            + jnp.dot(h2g, w1b_r[...], preferred_element_type=jnp.float32)
            + b1_r[...],
            0.0,
        )
        o_ref[...] = (
            jnp.dot(hh, w2_r[...], preferred_element_type=jnp.float32) + b2_r[...]
        )

    in_specs = [
        pl.BlockSpec((bn, C), lambda j: (j, 0)),
        pl.BlockSpec((2, bn, C), lambda j: (0, j, 0)),
        pl.BlockSpec((C, C), lambda j: (0, 0)),
        pl.BlockSpec((C, C), lambda j: (0, 0)),
        pl.BlockSpec((1, C), lambda j: (0, 0)),
        pl.BlockSpec((C, T), lambda j: (0, 0)),
        pl.BlockSpec((1, T), lambda j: (0, 0)),
    ]
    return pl.pallas_call(
        body,
        grid=(n // bn,),
        in_specs=in_specs,
        out_specs=pl.BlockSpec((bn, T), lambda j: (j, 0)),
        out_shape=jax.ShapeDtypeStruct((n, T), jnp.float32),
    )(h1, g, w1a, w1b, b1, w2, b2)


def _fold(p):
    wz = p["Wz"] @ p["lWz"][:C]
    bz = (p["bz"] @ p["lWz"][:C] + p["lbz"])[None]
    wh = p["Wh"] @ p["lWh"][:C]
    bh = (p["bh"] @ p["lWh"][:C] + p["lbh"])[None]
    probs = jnp.concatenate(
        [jax.nn.softmax(p["att"]), jnp.zeros((16 - T,), jnp.float32)]
    )[None]
    return wz, bz, wh, bh, probs


def kernel(x_1, edge_index_1, x_2, edge_index_2, address_start, address_end, params):
    NA1 = 10240
    NA2 = 2048
    NH1 = N1 + 16
    NH2 = N2 + 16

    iota1 = jnp.arange(N1, dtype=jnp.int32)
    src1, dst1 = edge_index_1[0], edge_index_1[1]
    src2, dst2 = edge_index_2[0], edge_index_2[1]

    d1 = _tc_colsum(_make_hist(_epad_hist(dst1.shape[0]), NH1)(dst1), N1, "dinv")
    d2 = _tc_colsum(_make_hist(_epad_hist(dst2.shape[0]), NH2)(dst2), N2, "dinv")

    ap = _pad_idx(address_start, _epad_hist(N1), N2)
    ep = _pad_idx(address_end, _epad_hist(N1), N2)
    ra = _tc_colsum(_make_hist(_epad_hist(N1), NH2)(ap), N2, "recip")
    re = _tc_colsum(_make_hist(_epad_hist(N1), NH2)(ep), N2, "recip")

    wz1, bz1, wh1, bh1, pr1 = _fold(params["main_1"])
    xt = jnp.transpose(x_1, (2, 0, 1))
    xts = _tc_scale3(xt, d1, 2000)
    plist = [_segsum(src1, dst1, xts[t], NA1, N1) for t in range(T)]
    h, stime = _tc_main1(xts, plist, d1, wz1, bz1, wh1, bh1, pr1, 400)

    time = x_1[:, -1, :]
    timep = jnp.concatenate([time, jnp.zeros((N1, C - T), jnp.float32)], axis=1)

    sa_h = _segsum(iota1, address_start, h, NA2, N2)
    se_h = _segsum(iota1, address_end, h, NA2, N2)
    sa_t = _segsum(iota1, address_start, timep, NA2, N2)
    se_t = _segsum(iota1, address_end, timep, NA2, N2)

    hs = _tc_scale2(h, d1, 2000)
    shp = _segsum(src1, dst1, hs, NA1, N1)

    tblA, tblB = _tc_groupcombine(sa_h, sa_t, se_h, se_t, ra, re, x_2, d2)
    pA = _segsum(src2, dst2, tblA, NA2, N2)
    pB = _segsum(src2, dst2, tblB, NA2, N2)
    wz3, bz3, wh3, bh3, pr3 = _fold(params["main_3"])
    fm = SECOND + C
    h2 = _tc_main3(
        pA, pB, tblA, tblB, d2,
        wz3[:fm], bz3, wh3[:fm], bh3,
        wz3[fm : fm + 1], wh3[fm : fm + 1], pr3, N2,
    )

    wz2, bz2, wh2, bh2, pr2 = _fold(params["main_2"])
    h1 = _tc_main2(
        shp, hs, stime, d1,
        wz2[:C], bz2, wh2[:C], bh2,
        wz2[C : C + 1], wh2[C : C + 1], pr2, 2000,
    )

    srcf = jnp.concatenate([address_start, address_end])
    dstf = jnp.concatenate([iota1, iota1])
    g = _segsum(srcf, dstf, h2, NA1, N1)
    pred = _tc_final(
        h1, g,
        params["W1"][:C], params["W1"][C:], params["b1"][None],
        params["W2"], params["b2"][None], 2000,
    )
    return pred


# speedup vs baseline: 1.9447x; 1.9447x over previous
import functools

import jax
import jax.numpy as jnp
from jax import lax
from jax.experimental import pallas as pl
from jax.experimental.pallas import tpu as pltpu
from jax.experimental.pallas import tpu_sc as plsc

N1 = 10000
N2 = 2000
T = 12
F0 = 128
C = 128
SECOND = 16

NTILES = 32
_K = 64


def _sc_mesh():
    return plsc.VectorSubcoreMesh(
        core_axis_name="c", subcore_axis_name="s", num_cores=2, num_subcores=16
    )


@functools.lru_cache(maxsize=None)
def _make_hist(e_pad, nh):
    per_tile = e_pad // NTILES
    assert per_tile % 16 == 0 and nh % 16 == 0

    @functools.partial(
        pl.kernel,
        out_type=jax.ShapeDtypeStruct((NTILES, nh), jnp.float32),
        mesh=_sc_mesh(),
        scratch_types=[
            pltpu.VMEM((per_tile,), jnp.int32),
            pltpu.VMEM((nh,), jnp.float32),
        ],
        compiler_params=pltpu.CompilerParams(needs_layout_passes=False),
    )
    def hist(lab_hbm, out_hbm, lab_v, hist_v):
        c = lax.axis_index("c")
        s = lax.axis_index("s")
        wid = c * 16 + s
        zvec = jnp.zeros((16,), jnp.float32)

        def zbody(i, carry):
            hist_v[pl.ds(i * 16, 16)] = zvec
            return carry

        lax.fori_loop(0, nh // 16, zbody, 0)
        pltpu.sync_copy(lab_hbm.at[pl.ds(wid * per_tile, per_tile)], lab_v)
        ones = jnp.full((16,), 1.0, jnp.float32)

        def body(i, carry):
            idx = lab_v[pl.ds(i * 16, 16)]
            plsc.addupdate_scatter(hist_v, [idx], ones)
            return carry

        lax.fori_loop(0, per_tile // 16, body, 0)
        pltpu.sync_copy(hist_v, out_hbm.at[wid])

    return hist


@functools.lru_cache(maxsize=None)
def _make_segsum(e_pad, n_acc, f):
    per_tile = e_pad // NTILES
    nchunks = per_tile // _K
    npairs = nchunks // 2
    rpt = n_acc // 16
    assert per_tile % (2 * _K) == 0 and rpt % 16 == 0 and f % 16 == 0

    @functools.partial(
        pl.kernel,
        out_type=jax.ShapeDtypeStruct((2, n_acc, f), jnp.float32),
        mesh=_sc_mesh(),
        scratch_types=[
            pltpu.VMEM((per_tile,), jnp.int32),
            pltpu.VMEM((per_tile,), jnp.int32),
            pltpu.VMEM((_K, f), jnp.float32),
            pltpu.VMEM((_K, f), jnp.float32),
            pltpu.VMEM((16, f), jnp.float32),
            pltpu.VMEM_SHARED((n_acc, f), jnp.float32),
            pltpu.SemaphoreType.DMA,
            pltpu.SemaphoreType.DMA,
        ],
    )
    def seg(src_hbm, dst_hbm, table_hbm, out_hbm, si_all, di_all,
            rows_a, rows_b, zb, acc, sem_a, sem_b):
        c = lax.axis_index("c")
        s = lax.axis_index("s")
        wid = c * 16 + s
        zvec = jnp.zeros((16,), jnp.float32)
        for r in range(16):
            for q in range(f // 16):
                zb[r, pl.ds(q * 16, 16)] = zvec

        base = wid * per_tile
        pltpu.sync_copy(src_hbm.at[pl.ds(base, per_tile)], si_all)
        pltpu.sync_copy(dst_hbm.at[pl.ds(base, per_tile)], di_all)
        pltpu.async_copy(table_hbm.at[si_all.at[pl.ds(0, _K)]], rows_a, sem_a)

        def zbody(i, carry):
            pltpu.sync_copy(zb, acc.at[pl.ds(s * rpt + i * 16, 16)])
            return carry

        lax.fori_loop(0, rpt // 16, zbody, 0)
        plsc.subcore_barrier()

        def pair(j, carry):
            i0 = 2 * j
            pltpu.async_copy(
                table_hbm.at[si_all.at[pl.ds((i0 + 1) * _K, _K)]], rows_b, sem_b
            )
            pltpu.make_async_copy(table_hbm.at[si_all.at[pl.ds(0, _K)]], rows_a, sem_a).wait()
            pltpu.sync_copy(rows_a, acc.at[di_all.at[pl.ds(i0 * _K, _K)]], add=True)
            inext = lax.rem(i0 + 2, nchunks)
            pltpu.async_copy(
                table_hbm.at[si_all.at[pl.ds(inext * _K, _K)]], rows_a, sem_a
            )
            pltpu.make_async_copy(table_hbm.at[si_all.at[pl.ds(0, _K)]], rows_b, sem_b).wait()
            pltpu.sync_copy(
                rows_b, acc.at[di_all.at[pl.ds((i0 + 1) * _K, _K)]], add=True
            )
            return carry

        lax.fori_loop(0, npairs, pair, 0)
        pltpu.make_async_copy(table_hbm.at[si_all.at[pl.ds(0, _K)]], rows_a, sem_a).wait()
        plsc.subcore_barrier()
        pltpu.sync_copy(
            acc.at[pl.ds(s * rpt, rpt)], out_hbm.at[c].at[pl.ds(s * rpt, rpt)]
        )

    return seg


def _pad_idx(a, e_pad, fill):
    return jnp.concatenate(
        [a, jnp.full((e_pad - a.shape[0],), fill, jnp.int32)]
    )


def _epad(e):
    blk = NTILES * _K * 2
    return ((e + blk - 1) // blk) * blk


def _epad_hist(e):
    per = NTILES * 16
    return ((e + per - 1) // per) * per


def _segsum(src, dst, table, n_acc, dummy):
    e = src.shape[0]
    e_pad = _epad(e)
    srcp = _pad_idx(src, e_pad, 0)
    dstp = _pad_idx(dst, e_pad, dummy)
    return _make_segsum(e_pad, n_acc, table.shape[1])(srcp, dstp, table)


def _tc_colsum(hist, n, mode):
    def body(h_ref, o_ref):
        ssum = jnp.sum(h_ref[...], axis=0, keepdims=True)
        if mode == "dinv":
            o_ref[...] = lax.rsqrt(ssum[:, :n] + 1.0)
        else:
            o_ref[...] = 1.0 / jnp.maximum(ssum[:, :n], 1e-12)

    out = pl.pallas_call(
        body, out_shape=jax.ShapeDtypeStruct((1, n), jnp.float32)
    )(hist)
    return out.reshape(n, 1)


def _tc_scale3(xt, d, bn):
    n = xt.shape[1]

    def body(x_ref, d_ref, o_ref):
        o_ref[...] = x_ref[...] * d_ref[...][None]

    return pl.pallas_call(
        body,
        grid=(T, n // bn),
        in_specs=[
            pl.BlockSpec((1, bn, F0), lambda t, j: (t, j, 0)),
            pl.BlockSpec((bn, 1), lambda t, j: (j, 0)),
        ],
        out_specs=pl.BlockSpec((1, bn, F0), lambda t, j: (t, j, 0)),
        out_shape=jax.ShapeDtypeStruct(xt.shape, jnp.float32),
    )(xt, d)


def _tc_scale2(tbl, d, bn):
    n, f = tbl.shape

    def body(x_ref, d_ref, o_ref):
        o_ref[...] = x_ref[...] * d_ref[...]

    return pl.pallas_call(
        body,
        grid=(n // bn,),
        in_specs=[
            pl.BlockSpec((bn, f), lambda j: (j, 0)),
            pl.BlockSpec((bn, 1), lambda j: (j, 0)),
        ],
        out_specs=pl.BlockSpec((bn, f), lambda j: (j, 0)),
        out_shape=jax.ShapeDtypeStruct((n, f), jnp.float32),
    )(tbl, d)


def _tc_main1(xts, plist, d, wz, bz, wh, bh, probs, bn):
    n = xts.shape[1]

    def body(*refs):
        x_ref = refs[0]
        p_refs = refs[1 : 1 + T]
        d_ref, wz_ref, bz_ref, wh_ref, bh_ref, pr_ref, o_ref, st_ref = refs[1 + T :]
        dv = d_ref[...]
        acc = jnp.zeros((bn, C), jnp.float32)
        cols = []
        for t in range(T):
            S = (p_refs[t][0] + p_refs[t][1] + x_ref[t]) * dv
            z = jax.nn.sigmoid(
                jnp.dot(S, wz_ref[...], preferred_element_type=jnp.float32)
                + bz_ref[...]
            )
            ht = jnp.tanh(
                jnp.dot(S, wh_ref[...], preferred_element_type=jnp.float32)
                + bh_ref[...]
            )
            acc = acc + pr_ref[0, t] * (1.0 - z) * ht
            cols.append(S[:, C - 1 : C])
        o_ref[...] = jnp.maximum(acc, 0.0)
        cols.append(jnp.zeros((bn, 16 - T), jnp.float32))
        st_ref[...] = jnp.concatenate(cols, axis=1)

    in_specs = [pl.BlockSpec((T, bn, F0), lambda j: (0, j, 0))]
    in_specs += [pl.BlockSpec((2, bn, C), lambda j: (0, j, 0)) for _ in range(T)]
    in_specs += [
        pl.BlockSpec((bn, 1), lambda j: (j, 0)),
        pl.BlockSpec((F0, C), lambda j: (0, 0)),
        pl.BlockSpec((1, C), lambda j: (0, 0)),
        pl.BlockSpec((F0, C), lambda j: (0, 0)),
        pl.BlockSpec((1, C), lambda j: (0, 0)),
        pl.BlockSpec((1, 16), lambda j: (0, 0)),
    ]
    return pl.pallas_call(
        body,
        grid=(n // bn,),
        in_specs=in_specs,
        out_specs=[
            pl.BlockSpec((bn, C), lambda j: (j, 0)),
            pl.BlockSpec((bn, 16), lambda j: (j, 0)),
        ],
        out_shape=[
            jax.ShapeDtypeStruct((n, C), jnp.float32),
            jax.ShapeDtypeStruct((n, 16), jnp.float32),
        ],
    )(xts, *plist, d, wz, bz, wh, bh, probs)


def _tc_main2(sp, tbl, st, d, wzh, bz, whh, bh, wzl, whl, probs, bn):
    n, f = tbl.shape

    def body(sp_ref, t_ref, st_ref, d_ref, wzh_r, bz_r, whh_r, bh_r,
             wzl_r, whl_r, pr_r, o_ref):
        Sh = (sp_ref[0] + sp_ref[1] + t_ref[...]) * d_ref[...]
        St = st_ref[...]
        Pz = jnp.dot(Sh, wzh_r[...], preferred_element_type=jnp.float32) + bz_r[...]
        Ph = jnp.dot(Sh, whh_r[...], preferred_element_type=jnp.float32) + bh_r[...]
        acc = jnp.zeros((bn, C), jnp.float32)
        for t in range(T):
            z = jax.nn.sigmoid(Pz + St[:, t : t + 1] * wzl_r[...])
            ht = jnp.tanh(Ph + St[:, t : t + 1] * whl_r[...])
            acc = acc + pr_r[0, t] * (1.0 - z) * ht
        o_ref[...] = jnp.maximum(acc, 0.0)

    in_specs = [
        pl.BlockSpec((2, bn, f), lambda j: (0, j, 0)),
        pl.BlockSpec((bn, f), lambda j: (j, 0)),
        pl.BlockSpec((bn, 16), lambda j: (j, 0)),
        pl.BlockSpec((bn, 1), lambda j: (j, 0)),
        pl.BlockSpec((C, C), lambda j: (0, 0)),
        pl.BlockSpec((1, C), lambda j: (0, 0)),
        pl.BlockSpec((C, C), lambda j: (0, 0)),
        pl.BlockSpec((1, C), lambda j: (0, 0)),
        pl.BlockSpec((1, C), lambda j: (0, 0)),
        pl.BlockSpec((1, C), lambda j: (0, 0)),
        pl.BlockSpec((1, 16), lambda j: (0, 0)),
    ]
    return pl.pallas_call(
        body,
        grid=(n // bn,),
        in_specs=in_specs,
        out_specs=pl.BlockSpec((bn, C), lambda j: (j, 0)),
        out_shape=jax.ShapeDtypeStruct((n, C), jnp.float32),
    )(sp, tbl, st, d, wzh, bz, whh, bh, wzl, whl, probs)


def _tc_main3(spA, spB, tblA, tblB, d, wzx, bz, whx, bh, wzl, whl, probs, bn):
    n = tblA.shape[0]

    def body(spA_r, spB_r, tA_r, tB_r, d_ref, wzx_r, bz_r, whx_r, bh_r,
             wzl_r, whl_r, pr_r, o_ref):
        dv = d_ref[...]
        SA = (spA_r[0] + spA_r[1] + tA_r[...]) * dv
        SB = (spB_r[0] + spB_r[1] + tB_r[...]) * dv
        Sh = jnp.concatenate([SA[:, :SECOND], SB], axis=1)
        St = SA[:, SECOND : SECOND + T]
        Pz = jnp.dot(Sh, wzx_r[...], preferred_element_type=jnp.float32) + bz_r[...]
        Ph = jnp.dot(Sh, whx_r[...], preferred_element_type=jnp.float32) + bh_r[...]
        acc = jnp.zeros((bn, C), jnp.float32)
        for t in range(T):
            z = jax.nn.sigmoid(Pz + St[:, t : t + 1] * wzl_r[...])
            ht = jnp.tanh(Ph + St[:, t : t + 1] * whl_r[...])
            acc = acc + pr_r[0, t] * (1.0 - z) * ht
        o_ref[...] = jnp.maximum(acc, 0.0)

    fm = SECOND + C
    in_specs = [
        pl.BlockSpec((2, bn, C), lambda j: (0, j, 0)),
        pl.BlockSpec((2, bn, C), lambda j: (0, j, 0)),
        pl.BlockSpec((bn, C), lambda j: (j, 0)),
        pl.BlockSpec((bn, C), lambda j: (j, 0)),
        pl.BlockSpec((bn, 1), lambda j: (j, 0)),
        pl.BlockSpec((fm, C), lambda j: (0, 0)),
        pl.BlockSpec((1, C), lambda j: (0, 0)),
        pl.BlockSpec((fm, C), lambda j: (0, 0)),
        pl.BlockSpec((1, C), lambda j: (0, 0)),
        pl.BlockSpec((1, C), lambda j: (0, 0)),
        pl.BlockSpec((1, C), lambda j: (0, 0)),
        pl.BlockSpec((1, 16), lambda j: (0, 0)),
    ]
    return pl.pallas_call(
        body,
        grid=(n // bn,),
        in_specs=in_specs,
        out_specs=pl.BlockSpec((bn, C), lambda j: (j, 0)),
        out_shape=jax.ShapeDtypeStruct((n, C), jnp.float32),
    )(spA, spB, tblA, tblB, d, wzx, bz, whx, bh, wzl, whl, probs)


def _tc_groupcombine(sa_h, sa_t, se_h, se_t, ra, re, x2, d2):
    def body(sah_r, sat_r, seh_r, set_r, ra_r, re_r, x2_r, d2_r, oa_ref, ob_ref):
        rav = ra_r[...]
        rev = re_r[...]
        gh = ((sah_r[0, :N2] + sah_r[1, :N2]) * rav
              + (seh_r[0, :N2] + seh_r[1, :N2]) * rev) * 0.5
        gt = ((sat_r[0, :N2] + sat_r[1, :N2]) * rav
              + (set_r[0, :N2] + set_r[1, :N2]) * rev) * 0.5
        d2v = d2_r[...]
        oa_ref[...] = (
            jnp.concatenate(
                [x2_r[...], gt[:, :T], jnp.zeros((N2, C - SECOND - T), jnp.float32)],
                axis=1,
            )
            * d2v
        )
        ob_ref[...] = gh * d2v

    return pl.pallas_call(
        body,
        out_shape=[
            jax.ShapeDtypeStruct((N2, C), jnp.float32),
            jax.ShapeDtypeStruct((N2, C), jnp.float32),
        ],
    )(sa_h, sa_t, se_h, se_t, ra, re, x2, d2)


def _tc_final(h1, g, w1a, w1b, b1, w2, b2, bn):
    n = h1.shape[0]

    def body(h1_r, g_r, w1a_r, w1b_r, b1_r, w2_r, b2_r, o_ref):
        h2g = (g_r[0] + g_r[1]) * 0.5
        hh = jnp.maximum(
            jnp.dot(h1_r[...], w1a_r[...], preferred_element_type=jnp.float32)
            + jnp.dot(h2g, w1b_r[...], preferred_element_type=jnp.float32)
            + b1_r[...],
            0.0,
        )
        o_ref[...] = (
            jnp.dot(hh, w2_r[...], preferred_element_type=jnp.float32) + b2_r[...]
        )

    in_specs = [
        pl.BlockSpec((bn, C), lambda j: (j, 0)),
        pl.BlockSpec((2, bn, C), lambda j: (0, j, 0)),
        pl.BlockSpec((C, C), lambda j: (0, 0)),
        pl.BlockSpec((C, C), lambda j: (0, 0)),
        pl.BlockSpec((1, C), lambda j: (0, 0)),
        pl.BlockSpec((C, T), lambda j: (0, 0)),
        pl.BlockSpec((1, T), lambda j: (0, 0)),
    ]
    return pl.pallas_call(
        body,
        grid=(n // bn,),
        in_specs=in_specs,
        out_specs=pl.BlockSpec((bn, T), lambda j: (j, 0)),
        out_shape=jax.ShapeDtypeStruct((n, T), jnp.float32),
    )(h1, g, w1a, w1b, b1, w2, b2)


def _fold(p):
    wz = p["Wz"] @ p["lWz"][:C]
    bz = (p["bz"] @ p["lWz"][:C] + p["lbz"])[None]
    wh = p["Wh"] @ p["lWh"][:C]
    bh = (p["bh"] @ p["lWh"][:C] + p["lbh"])[None]
    probs = jnp.concatenate(
        [jax.nn.softmax(p["att"]), jnp.zeros((16 - T,), jnp.float32)]
    )[None]
    return wz, bz, wh, bh, probs


def kernel(x_1, edge_index_1, x_2, edge_index_2, address_start, address_end, params):
    NA1 = 10240
    NA2 = 2048
    NH1 = N1 + 16
    NH2 = N2 + 16

    iota1 = jnp.arange(N1, dtype=jnp.int32)
    src1, dst1 = edge_index_1[0], edge_index_1[1]
    src2, dst2 = edge_index_2[0], edge_index_2[1]

    d1 = _tc_colsum(_make_hist(_epad_hist(dst1.shape[0]), NH1)(dst1), N1, "dinv")
    d2 = _tc_colsum(_make_hist(_epad_hist(dst2.shape[0]), NH2)(dst2), N2, "dinv")

    ap = _pad_idx(address_start, _epad_hist(N1), N2)
    ep = _pad_idx(address_end, _epad_hist(N1), N2)
    ra = _tc_colsum(_make_hist(_epad_hist(N1), NH2)(ap), N2, "recip")
    re = _tc_colsum(_make_hist(_epad_hist(N1), NH2)(ep), N2, "recip")

    wz1, bz1, wh1, bh1, pr1 = _fold(params["main_1"])
    xt = jnp.transpose(x_1, (2, 0, 1))
    xts = _tc_scale3(xt, d1, 2000)
    plist = [_segsum(src1, dst1, xts[t], NA1, N1) for t in range(T)]
    h, stime = _tc_main1(xts, plist, d1, wz1, bz1, wh1, bh1, pr1, 400)

    time = x_1[:, -1, :]
    timep = jnp.concatenate([time, jnp.zeros((N1, C - T), jnp.float32)], axis=1)

    sa_h = _segsum(iota1, address_start, h, NA2, N2)
    se_h = _segsum(iota1, address_end, h, NA2, N2)
    sa_t = _segsum(iota1, address_start, timep, NA2, N2)
    se_t = _segsum(iota1, address_end, timep, NA2, N2)

    hs = _tc_scale2(h, d1, 2000)
    shp = _segsum(src1, dst1, hs, NA1, N1)

    tblA, tblB = _tc_groupcombine(sa_h, sa_t, se_h, se_t, ra, re, x_2, d2)
    pA = _segsum(src2, dst2, tblA, NA2, N2)
    pB = _segsum(src2, dst2, tblB, NA2, N2)
    wz3, bz3, wh3, bh3, pr3 = _fold(params["main_3"])
    fm = SECOND + C
    h2 = _tc_main3(
        pA, pB, tblA, tblB, d2,
        wz3[:fm], bz3, wh3[:fm], bh3,
        wz3[fm : fm + 1], wh3[fm : fm + 1], pr3, N2,
    )

    wz2, bz2, wh2, bh2, pr2 = _fold(params["main_2"])
    h1 = _tc_main2(
        shp, hs, stime, d1,
        wz2[:C], bz2, wh2[:C], bh2,
        wz2[C : C + 1], wh2[C : C + 1], pr2, 2000,
    )

    srcf = jnp.concatenate([address_start, address_end])
    dstf = jnp.concatenate([iota1, iota1])
    g = _segsum(srcf, dstf, h2, NA1, N1)
    pred = _tc_final(
        h1, g,
        params["W1"][:C], params["W1"][C:], params["b1"][None],
        params["W2"], params["b2"][None], 2000,
    )
    return pred


# speedup vs baseline: 2.6917x; 1.3841x over previous
import functools

import jax
import jax.numpy as jnp
from jax import lax
from jax.experimental import pallas as pl
from jax.experimental.pallas import tpu as pltpu
from jax.experimental.pallas import tpu_sc as plsc

N1 = 10000
N2 = 2000
T = 12
F0 = 128
C = 128
SECOND = 16

NTILES = 32
_K = 80


def _sc_mesh():
    return plsc.VectorSubcoreMesh(
        core_axis_name="c", subcore_axis_name="s", num_cores=2, num_subcores=16
    )


@functools.lru_cache(maxsize=None)
def _make_hist(e_pad, nh):
    per_tile = e_pad // NTILES
    assert per_tile % 16 == 0 and nh % 16 == 0

    @functools.partial(
        pl.kernel,
        out_type=jax.ShapeDtypeStruct((NTILES, nh), jnp.float32),
        mesh=_sc_mesh(),
        scratch_types=[
            pltpu.VMEM((per_tile,), jnp.int32),
            pltpu.VMEM((nh,), jnp.float32),
        ],
        compiler_params=pltpu.CompilerParams(needs_layout_passes=False),
    )
    def hist(lab_hbm, out_hbm, lab_v, hist_v):
        c = lax.axis_index("c")
        s = lax.axis_index("s")
        wid = c * 16 + s
        zvec = jnp.zeros((16,), jnp.float32)

        def zbody(i, carry):
            hist_v[pl.ds(i * 16, 16)] = zvec
            return carry

        lax.fori_loop(0, nh // 16, zbody, 0)
        pltpu.sync_copy(lab_hbm.at[pl.ds(wid * per_tile, per_tile)], lab_v)
        ones = jnp.full((16,), 1.0, jnp.float32)

        def body(i, carry):
            idx = lab_v[pl.ds(i * 16, 16)]
            plsc.addupdate_scatter(hist_v, [idx], ones)
            return carry

        lax.fori_loop(0, per_tile // 16, body, 0)
        pltpu.sync_copy(hist_v, out_hbm.at[wid])

    return hist


@functools.lru_cache(maxsize=None)
def _make_segsum(e_pad, n_acc, f):
    per_tile = e_pad // NTILES
    nchunks = per_tile // _K
    npairs = nchunks // 2
    rpt = n_acc // 16
    assert per_tile % (2 * _K) == 0 and rpt % 32 == 0 and f % 16 == 0

    @functools.partial(
        pl.kernel,
        out_type=jax.ShapeDtypeStruct((2, n_acc, f), jnp.float32),
        mesh=_sc_mesh(),
        scratch_types=[
            pltpu.VMEM((per_tile,), jnp.int32),
            pltpu.VMEM((per_tile,), jnp.int32),
            pltpu.VMEM((_K, f), jnp.float32),
            pltpu.VMEM((_K, f), jnp.float32),
            pltpu.VMEM((32, f), jnp.float32),
            pltpu.VMEM_SHARED((n_acc, f), jnp.float32),
            pltpu.SemaphoreType.DMA,
            pltpu.SemaphoreType.DMA,
        ],
    )
    def seg(src_hbm, dst_hbm, table_hbm, out_hbm, si_all, di_all,
            rows_a, rows_b, zb, acc, sem_a, sem_b):
        c = lax.axis_index("c")
        s = lax.axis_index("s")
        wid = c * 16 + s
        zvec = jnp.zeros((16,), jnp.float32)
        for r in range(32):
            for q in range(f // 16):
                zb[r, pl.ds(q * 16, 16)] = zvec

        base = wid * per_tile
        pltpu.sync_copy(src_hbm.at[pl.ds(base, per_tile)], si_all)
        pltpu.sync_copy(dst_hbm.at[pl.ds(base, per_tile)], di_all)
        pltpu.async_copy(table_hbm.at[si_all.at[pl.ds(0, _K)]], rows_a, sem_a)

        def zbody(i, carry):
            pltpu.sync_copy(zb, acc.at[pl.ds(s * rpt + i * 32, 32)])
            return carry

        lax.fori_loop(0, rpt // 32, zbody, 0)
        plsc.subcore_barrier()

        def pair(j, carry):
            i0 = 2 * j
            pltpu.async_copy(
                table_hbm.at[si_all.at[pl.ds((i0 + 1) * _K, _K)]], rows_b, sem_b
            )
            pltpu.make_async_copy(table_hbm.at[si_all.at[pl.ds(0, _K)]], rows_a, sem_a).wait()
            pltpu.sync_copy(rows_a, acc.at[di_all.at[pl.ds(i0 * _K, _K)]], add=True)
            inext = lax.rem(i0 + 2, nchunks)
            pltpu.async_copy(
                table_hbm.at[si_all.at[pl.ds(inext * _K, _K)]], rows_a, sem_a
            )
            pltpu.make_async_copy(table_hbm.at[si_all.at[pl.ds(0, _K)]], rows_b, sem_b).wait()
            pltpu.sync_copy(
                rows_b, acc.at[di_all.at[pl.ds((i0 + 1) * _K, _K)]], add=True
            )
            return carry

        lax.fori_loop(0, npairs, pair, 0)
        pltpu.make_async_copy(table_hbm.at[si_all.at[pl.ds(0, _K)]], rows_a, sem_a).wait()
        plsc.subcore_barrier()
        pltpu.sync_copy(
            acc.at[pl.ds(s * rpt, rpt)], out_hbm.at[c].at[pl.ds(s * rpt, rpt)]
        )

    return seg


def _pad_idx(a, e_pad, fill):
    return jnp.concatenate(
        [a, jnp.full((e_pad - a.shape[0],), fill, jnp.int32)]
    )


def _epad(e):
    blk = NTILES * _K * 2
    return ((e + blk - 1) // blk) * blk


def _epad_hist(e):
    per = NTILES * 16
    return ((e + per - 1) // per) * per


def _segsum(src, dst, table, n_acc, dummy):
    e = src.shape[0]
    e_pad = _epad(e)
    srcp = _pad_idx(src, e_pad, 0)
    dstp = _pad_idx(dst, e_pad, dummy)
    return _make_segsum(e_pad, n_acc, table.shape[1])(srcp, dstp, table)


def _tc_colsum(hist, n, mode):
    def body(h_ref, o_ref):
        ssum = jnp.sum(h_ref[...], axis=0, keepdims=True)
        if mode == "dinv":
            o_ref[...] = lax.rsqrt(ssum[:, :n] + 1.0)
        else:
            o_ref[...] = 1.0 / jnp.maximum(ssum[:, :n], 1e-12)

    out = pl.pallas_call(
        body, out_shape=jax.ShapeDtypeStruct((1, n), jnp.float32)
    )(hist)
    return out.reshape(n, 1)


def _tc_scale3(xt, d, bn):
    n = xt.shape[1]

    def body(x_ref, d_ref, o_ref):
        o_ref[...] = x_ref[...] * d_ref[...][None]

    return pl.pallas_call(
        body,
        grid=(T, n // bn),
        in_specs=[
            pl.BlockSpec((1, bn, F0), lambda t, j: (t, j, 0)),
            pl.BlockSpec((bn, 1), lambda t, j: (j, 0)),
        ],
        out_specs=pl.BlockSpec((1, bn, F0), lambda t, j: (t, j, 0)),
        out_shape=jax.ShapeDtypeStruct(xt.shape, jnp.float32),
    )(xt, d)


def _tc_scale2(tbl, d, bn):
    n, f = tbl.shape

    def body(x_ref, d_ref, o_ref):
        o_ref[...] = x_ref[...] * d_ref[...]

    return pl.pallas_call(
        body,
        grid=(n // bn,),
        in_specs=[
            pl.BlockSpec((bn, f), lambda j: (j, 0)),
            pl.BlockSpec((bn, 1), lambda j: (j, 0)),
        ],
        out_specs=pl.BlockSpec((bn, f), lambda j: (j, 0)),
        out_shape=jax.ShapeDtypeStruct((n, f), jnp.float32),
    )(tbl, d)


def _tc_main1(xts, plist, d, wz, bz, wh, bh, probs, bn):
    n = xts.shape[1]

    def body(*refs):
        x_ref = refs[0]
        p_refs = refs[1 : 1 + T]
        d_ref, wz_ref, bz_ref, wh_ref, bh_ref, pr_ref, o_ref, st_ref = refs[1 + T :]
        dv = d_ref[...]
        acc = jnp.zeros((bn, C), jnp.float32)
        cols = []
        for t in range(T):
            S = (p_refs[t][0] + p_refs[t][1] + x_ref[t]) * dv
            z = jax.nn.sigmoid(
                jnp.dot(S, wz_ref[...], preferred_element_type=jnp.float32)
                + bz_ref[...]
            )
            ht = jnp.tanh(
                jnp.dot(S, wh_ref[...], preferred_element_type=jnp.float32)
                + bh_ref[...]
            )
            acc = acc + pr_ref[0, t] * (1.0 - z) * ht
            cols.append(S[:, C - 1 : C])
        o_ref[...] = jnp.maximum(acc, 0.0)
        cols.append(jnp.zeros((bn, 16 - T), jnp.float32))
        st_ref[...] = jnp.concatenate(cols, axis=1)

    in_specs = [pl.BlockSpec((T, bn, F0), lambda j: (0, j, 0))]
    in_specs += [pl.BlockSpec((2, bn, C), lambda j: (0, j, 0)) for _ in range(T)]
    in_specs += [
        pl.BlockSpec((bn, 1), lambda j: (j, 0)),
        pl.BlockSpec((F0, C), lambda j: (0, 0)),
        pl.BlockSpec((1, C), lambda j: (0, 0)),
        pl.BlockSpec((F0, C), lambda j: (0, 0)),
        pl.BlockSpec((1, C), lambda j: (0, 0)),
        pl.BlockSpec((1, 16), lambda j: (0, 0)),
    ]
    return pl.pallas_call(
        body,
        grid=(n // bn,),
        in_specs=in_specs,
        out_specs=[
            pl.BlockSpec((bn, C), lambda j: (j, 0)),
            pl.BlockSpec((bn, 16), lambda j: (j, 0)),
        ],
        out_shape=[
            jax.ShapeDtypeStruct((n, C), jnp.float32),
            jax.ShapeDtypeStruct((n, 16), jnp.float32),
        ],
    )(xts, *plist, d, wz, bz, wh, bh, probs)


def _tc_main2(sp, tbl, st, d, wzh, bz, whh, bh, wzl, whl, probs, bn):
    n, f = tbl.shape

    def body(sp_ref, t_ref, st_ref, d_ref, wzh_r, bz_r, whh_r, bh_r,
             wzl_r, whl_r, pr_r, o_ref):
        Sh = (sp_ref[0] + sp_ref[1] + t_ref[...]) * d_ref[...]
        St = st_ref[...]
        Pz = jnp.dot(Sh, wzh_r[...], preferred_element_type=jnp.float32) + bz_r[...]
        Ph = jnp.dot(Sh, whh_r[...], preferred_element_type=jnp.float32) + bh_r[...]
        acc = jnp.zeros((bn, C), jnp.float32)
        for t in range(T):
            z = jax.nn.sigmoid(Pz + St[:, t : t + 1] * wzl_r[...])
            ht = jnp.tanh(Ph + St[:, t : t + 1] * whl_r[...])
            acc = acc + pr_r[0, t] * (1.0 - z) * ht
        o_ref[...] = jnp.maximum(acc, 0.0)

    in_specs = [
        pl.BlockSpec((2, bn, f), lambda j: (0, j, 0)),
        pl.BlockSpec((bn, f), lambda j: (j, 0)),
        pl.BlockSpec((bn, 16), lambda j: (j, 0)),
        pl.BlockSpec((bn, 1), lambda j: (j, 0)),
        pl.BlockSpec((C, C), lambda j: (0, 0)),
        pl.BlockSpec((1, C), lambda j: (0, 0)),
        pl.BlockSpec((C, C), lambda j: (0, 0)),
        pl.BlockSpec((1, C), lambda j: (0, 0)),
        pl.BlockSpec((1, C), lambda j: (0, 0)),
        pl.BlockSpec((1, C), lambda j: (0, 0)),
        pl.BlockSpec((1, 16), lambda j: (0, 0)),
    ]
    return pl.pallas_call(
        body,
        grid=(n // bn,),
        in_specs=in_specs,
        out_specs=pl.BlockSpec((bn, C), lambda j: (j, 0)),
        out_shape=jax.ShapeDtypeStruct((n, C), jnp.float32),
    )(sp, tbl, st, d, wzh, bz, whh, bh, wzl, whl, probs)


def _tc_main3(spA, spB, tblA, tblB, d, wzx, bz, whx, bh, wzl, whl, probs, bn):
    n = tblA.shape[0]

    def body(spA_r, spB_r, tA_r, tB_r, d_ref, wzx_r, bz_r, whx_r, bh_r,
             wzl_r, whl_r, pr_r, o_ref):
        dv = d_ref[...]
        SA = (spA_r[0] + spA_r[1] + tA_r[...]) * dv
        SB = (spB_r[0] + spB_r[1] + tB_r[...]) * dv
        Sh = jnp.concatenate([SA[:, :SECOND], SB], axis=1)
        St = SA[:, SECOND : SECOND + T]
        Pz = jnp.dot(Sh, wzx_r[...], preferred_element_type=jnp.float32) + bz_r[...]
        Ph = jnp.dot(Sh, whx_r[...], preferred_element_type=jnp.float32) + bh_r[...]
        acc = jnp.zeros((bn, C), jnp.float32)
        for t in range(T):
            z = jax.nn.sigmoid(Pz + St[:, t : t + 1] * wzl_r[...])
            ht = jnp.tanh(Ph + St[:, t : t + 1] * whl_r[...])
            acc = acc + pr_r[0, t] * (1.0 - z) * ht
        o_ref[...] = jnp.maximum(acc, 0.0)

    fm = SECOND + C
    in_specs = [
        pl.BlockSpec((2, bn, C), lambda j: (0, j, 0)),
        pl.BlockSpec((2, bn, C), lambda j: (0, j, 0)),
        pl.BlockSpec((bn, C), lambda j: (j, 0)),
        pl.BlockSpec((bn, C), lambda j: (j, 0)),
        pl.BlockSpec((bn, 1), lambda j: (j, 0)),
        pl.BlockSpec((fm, C), lambda j: (0, 0)),
        pl.BlockSpec((1, C), lambda j: (0, 0)),
        pl.BlockSpec((fm, C), lambda j: (0, 0)),
        pl.BlockSpec((1, C), lambda j: (0, 0)),
        pl.BlockSpec((1, C), lambda j: (0, 0)),
        pl.BlockSpec((1, C), lambda j: (0, 0)),
        pl.BlockSpec((1, 16), lambda j: (0, 0)),
    ]
    return pl.pallas_call(
        body,
        grid=(n // bn,),
        in_specs=in_specs,
        out_specs=pl.BlockSpec((bn, C), lambda j: (j, 0)),
        out_shape=jax.ShapeDtypeStruct((n, C), jnp.float32),
    )(spA, spB, tblA, tblB, d, wzx, bz, whx, bh, wzl, whl, probs)


def _tc_groupcombine(sa_h, sa_t, se_h, se_t, ra, re, x2, d2):
    def body(sah_r, sat_r, seh_r, set_r, ra_r, re_r, x2_r, d2_r, oa_ref, ob_ref):
        rav = ra_r[...]
        rev = re_r[...]
        gh = ((sah_r[0, :N2] + sah_r[1, :N2]) * rav
              + (seh_r[0, :N2] + seh_r[1, :N2]) * rev) * 0.5
        gt = ((sat_r[0, :N2] + sat_r[1, :N2]) * rav
              + (set_r[0, :N2] + set_r[1, :N2]) * rev) * 0.5
        d2v = d2_r[...]
        oa_ref[...] = (
            jnp.concatenate(
                [x2_r[...], gt[:, :T], jnp.zeros((N2, C - SECOND - T), jnp.float32)],
                axis=1,
            )
            * d2v
        )
        ob_ref[...] = gh * d2v

    return pl.pallas_call(
        body,
        out_shape=[
            jax.ShapeDtypeStruct((N2, C), jnp.float32),
            jax.ShapeDtypeStruct((N2, C), jnp.float32),
        ],
    )(sa_h, sa_t, se_h, se_t, ra, re, x2, d2)


def _tc_final(h1, g, w1a, w1b, b1, w2, b2, bn):
    n = h1.shape[0]

    def body(h1_r, g_r, w1a_r, w1b_r, b1_r, w2_r, b2_r, o_ref):
        h2g = (g_r[0] + g_r[1]) * 0.5
        hh = jnp.maximum(
            jnp.dot(h1_r[...], w1a_r[...], preferred_element_type=jnp.float32)
            + jnp.dot(h2g, w1b_r[...], preferred_element_type=jnp.float32)
            + b1_r[...],
            0.0,
        )
        o_ref[...] = (
            jnp.dot(hh, w2_r[...], preferred_element_type=jnp.float32) + b2_r[...]
        )

    in_specs = [
        pl.BlockSpec((bn, C), lambda j: (j, 0)),
        pl.BlockSpec((2, bn, C), lambda j: (0, j, 0)),
        pl.BlockSpec((C, C), lambda j: (0, 0)),
        pl.BlockSpec((C, C), lambda j: (0, 0)),
        pl.BlockSpec((1, C), lambda j: (0, 0)),
        pl.BlockSpec((C, T), lambda j: (0, 0)),
        pl.BlockSpec((1, T), lambda j: (0, 0)),
    ]
    return pl.pallas_call(
        body,
        grid=(n // bn,),
        in_specs=in_specs,
        out_specs=pl.BlockSpec((bn, T), lambda j: (j, 0)),
        out_shape=jax.ShapeDtypeStruct((n, T), jnp.float32),
    )(h1, g, w1a, w1b, b1, w2, b2)


def _fold(p):
    wz = p["Wz"] @ p["lWz"][:C]
    bz = (p["bz"] @ p["lWz"][:C] + p["lbz"])[None]
    wh = p["Wh"] @ p["lWh"][:C]
    bh = (p["bh"] @ p["lWh"][:C] + p["lbh"])[None]
    probs = jnp.concatenate(
        [jax.nn.softmax(p["att"]), jnp.zeros((16 - T,), jnp.float32)]
    )[None]
    return wz, bz, wh, bh, probs


def kernel(x_1, edge_index_1, x_2, edge_index_2, address_start, address_end, params):
    NA1 = 10240
    NA2 = 2048
    NH1 = N1 + 16
    NH2 = N2 + 16

    iota1 = jnp.arange(N1, dtype=jnp.int32)
    src1, dst1 = edge_index_1[0], edge_index_1[1]
    src2, dst2 = edge_index_2[0], edge_index_2[1]

    d1 = _tc_colsum(_make_hist(_epad_hist(dst1.shape[0]), NH1)(dst1), N1, "dinv")
    d2 = _tc_colsum(_make_hist(_epad_hist(dst2.shape[0]), NH2)(dst2), N2, "dinv")

    ap = _pad_idx(address_start, _epad_hist(N1), N2)
    ep = _pad_idx(address_end, _epad_hist(N1), N2)
    ra = _tc_colsum(_make_hist(_epad_hist(N1), NH2)(ap), N2, "recip")
    re = _tc_colsum(_make_hist(_epad_hist(N1), NH2)(ep), N2, "recip")

    wz1, bz1, wh1, bh1, pr1 = _fold(params["main_1"])
    xt = jnp.transpose(x_1, (2, 0, 1))
    xts = _tc_scale3(xt, d1, 2000)
    plist = [_segsum(src1, dst1, xts[t], NA1, N1) for t in range(T)]
    h, stime = _tc_main1(xts, plist, d1, wz1, bz1, wh1, bh1, pr1, 400)

    time = x_1[:, -1, :]
    timep = jnp.concatenate([time, jnp.zeros((N1, C - T), jnp.float32)], axis=1)

    sa_h = _segsum(iota1, address_start, h, NA2, N2)
    se_h = _segsum(iota1, address_end, h, NA2, N2)
    sa_t = _segsum(iota1, address_start, timep, NA2, N2)
    se_t = _segsum(iota1, address_end, timep, NA2, N2)

    hs = _tc_scale2(h, d1, 2000)
    shp = _segsum(src1, dst1, hs, NA1, N1)

    tblA, tblB = _tc_groupcombine(sa_h, sa_t, se_h, se_t, ra, re, x_2, d2)
    pA = _segsum(src2, dst2, tblA, NA2, N2)
    pB = _segsum(src2, dst2, tblB, NA2, N2)
    wz3, bz3, wh3, bh3, pr3 = _fold(params["main_3"])
    fm = SECOND + C
    h2 = _tc_main3(
        pA, pB, tblA, tblB, d2,
        wz3[:fm], bz3, wh3[:fm], bh3,
        wz3[fm : fm + 1], wh3[fm : fm + 1], pr3, N2,
    )

    wz2, bz2, wh2, bh2, pr2 = _fold(params["main_2"])
    h1 = _tc_main2(
        shp, hs, stime, d1,
        wz2[:C], bz2, wh2[:C], bh2,
        wz2[C : C + 1], wh2[C : C + 1], pr2, 2000,
    )

    srcf = jnp.concatenate([address_start, address_end])
    dstf = jnp.concatenate([iota1, iota1])
    g = _segsum(srcf, dstf, h2, NA1, N1)
    pred = _tc_final(
        h1, g,
        params["W1"][:C], params["W1"][C:], params["b1"][None],
        params["W2"], params["b2"][None], 2000,
    )
    return pred
